# Initial kernel scaffold; baseline (speedup 1.0000x reference)
import jax
import jax.numpy as jnp
from jax.experimental import pallas as pl


def kernel(x, edge_index, edge_attr, batch, W1s, b1s, W2s, b2s, gammas, betas, pred_W, pred_b):
    raise NotImplementedError("write your pallas kernel here")


# speedup vs baseline: 2.4827x; 2.4827x over previous
import functools

import jax
import jax.numpy as jnp
from jax import lax
from jax.experimental import pallas as pl
from jax.experimental.pallas import tpu as pltpu
from jax.experimental.pallas import tpu_sc as plsc

N = 10000
E = 320000
D = 128
NUM_LAYERS = 4
NUM_GRAPHS = 64
NUM_TASKS = 10
BN_EPS = 1e-5

NC = 2
NS = 16
NT = NC * NS
W = 240
ACC = 528
CH = 128


def _tile_bounds():
    per_sc = E // NC
    nwin = -(-per_sc // W)
    base, extra = divmod(nwin, NS)
    bounds = []
    for sc in range(NC):
        lo = sc * per_sc
        for t in range(NS):
            wins = base + (1 if t < extra else 0)
            hi = min(lo + wins * W, (sc + 1) * per_sc)
            bounds.append((lo, hi - lo))
            lo = hi
    return bounds

_BOUNDS = _tile_bounds()


def _sel(t, vals):
    r = jnp.int32(vals[0])
    for i in range(1, NT):
        r = jnp.where(t == i, jnp.int32(vals[i]), r)
    return r


def _agg_body(h_hbm, src_hbm, dst_hbm, parts_hbm, bases_hbm,
              idxv, dstv, rows, acc, basev, sem):
    c = lax.axis_index("c")
    s = lax.axis_index("s")
    t = c * NS + s
    lo = pl.multiple_of(_sel(t, [b[0] for b in _BOUNDS]), 16)
    cnt = _sel(t, [b[1] for b in _BOUNDS])

    pltpu.sync_copy(dst_hbm.at[pl.ds(lo, 16)], basev)
    bvec = basev[...]
    base = (bvec[0] // 8) * 8
    basev[...] = jnp.full((16,), base, jnp.int32)

    def zb(i, carry):
        for k in range(8):
            acc[i, pl.ds(k * 16, 16)] = jnp.zeros((16,), jnp.float32)
        return carry
    lax.fori_loop(0, ACC, zb, 0)

    nch = (cnt + CH - 1) // CH

    def chunk(ci, carry):
        off = pl.multiple_of(lo + ci * CH, 16)
        pltpu.sync_copy(src_hbm.at[pl.ds(off, CH)], idxv)
        pltpu.sync_copy(dst_hbm.at[pl.ds(off, CH)], dstv)
        pltpu.async_copy(h_hbm.at[idxv], rows, sem).wait()
        ne = jnp.minimum(cnt - ci * CH, CH)
        ngrp = (ne + 15) // 16

        def grp(g, gcarry):
            dvec = jnp.minimum(dstv[pl.ds(g * 16, 16)] - base, ACC - 1)
            nj = jnp.minimum(ne - g * 16, 16)

            for j16 in range(16):
                d = dvec[j16]

                @pl.when(j16 < nj)
                def _add():
                    r = g * 16 + j16
                    for k in range(8):
                        plsc.addupdate(acc.at[d, pl.ds(k * 16, 16)],
                                       rows[r, pl.ds(k * 16, 16)])
            return gcarry
        lax.fori_loop(0, ngrp, grp, 0)
        return carry
    lax.fori_loop(0, nch, chunk, 0)

    pltpu.sync_copy(acc, parts_hbm.at[t])
    pltpu.sync_copy(basev, bases_hbm.at[t])


@functools.cache
def _make_agg_call():
    return functools.partial(
        pl.kernel,
        out_type=(jax.ShapeDtypeStruct((NT, ACC, D), jnp.float32),
                  jax.ShapeDtypeStruct((NT, 16), jnp.int32)),
        mesh=plsc.VectorSubcoreMesh(core_axis_name="c", subcore_axis_name="s"),
        scratch_types=[
            pltpu.VMEM((CH,), jnp.int32),
            pltpu.VMEM((CH,), jnp.int32),
            pltpu.VMEM((CH, D), jnp.float32),
            pltpu.VMEM((ACC, D), jnp.float32),
            pltpu.VMEM((16,), jnp.int32),
            pltpu.SemaphoreType.DMA,
        ],
    )(_agg_body)


def _agg_call(h, ssrc, sdst):
    return _make_agg_call()(h, ssrc, sdst)


NPAD = N + ACC + 8


def _mlp_body(h_ref, parts_ref, bases_ref, w1_ref, b1_ref, w2_ref, b2_ref,
              out_ref, agg_ref):
    agg_ref[...] = jnp.zeros((NPAD, D), jnp.float32)
    for t in range(NT):
        b = bases_ref[t, 0]
        blk = agg_ref[pl.ds(b, ACC), :]
        agg_ref[pl.ds(b, ACC), :] = blk + parts_ref[t]
    z = h_ref[...] + agg_ref[0:N, :]
    z = jnp.dot(z, w1_ref[...], preferred_element_type=jnp.float32) + b1_ref[...]
    z = jnp.maximum(z, 0.0)
    z = jnp.dot(z, w2_ref[...], preferred_element_type=jnp.float32) + b2_ref[...]
    out_ref[...] = z


_mlp_call = pl.pallas_call(
    _mlp_body,
    out_shape=jax.ShapeDtypeStruct((N, D), jnp.float32),
    in_specs=[
        pl.BlockSpec(memory_space=pltpu.VMEM),
        pl.BlockSpec(memory_space=pltpu.VMEM),
        pl.BlockSpec(memory_space=pltpu.SMEM),
        pl.BlockSpec(memory_space=pltpu.VMEM),
        pl.BlockSpec(memory_space=pltpu.VMEM),
        pl.BlockSpec(memory_space=pltpu.VMEM),
        pl.BlockSpec(memory_space=pltpu.VMEM),
    ],
    scratch_shapes=[pltpu.VMEM((NPAD, D), jnp.float32)],
)


def _bn_call(z, mean, var, gamma, beta, last):
    def body(z_ref, m_ref, v_ref, g_ref, be_ref, out_ref):
        zz = (z_ref[...] - m_ref[...]) / jnp.sqrt(v_ref[...] + BN_EPS) \
            * g_ref[...] + be_ref[...]
        if not last:
            zz = jnp.maximum(zz, 0.0)
        out_ref[...] = zz

    return pl.pallas_call(
        body,
        out_shape=jax.ShapeDtypeStruct((N, D), jnp.float32),
    )(z, mean, var, gamma, beta)


def _head_body(h_ref, batch_ref, pw_ref, pb_ref, out_ref):
    onehot = (lax.broadcasted_iota(jnp.int32, (NUM_GRAPHS, N), 0)
              == batch_ref[...]).astype(jnp.float32)
    g = jnp.dot(onehot, h_ref[...], preferred_element_type=jnp.float32,
                precision=lax.Precision.HIGHEST)
    o = jnp.dot(g, pw_ref[...], preferred_element_type=jnp.float32) + pb_ref[...]
    out_ref[...] = jnp.where(o > 0, o, 0.1 * o)


_head_call = pl.pallas_call(
    _head_body,
    out_shape=jax.ShapeDtypeStruct((NUM_GRAPHS, NUM_TASKS), jnp.float32),
)


def kernel(x, edge_index, edge_attr, batch, W1s, b1s, W2s, b2s, gammas, betas, pred_W, pred_b):
    src = edge_index[0].astype(jnp.int32)
    dst = edge_index[1].astype(jnp.int32)
    order = jnp.argsort(dst, stable=True)
    ssrc = jnp.concatenate([src[order], jnp.zeros((CH,), jnp.int32)])
    sdst = jnp.concatenate([dst[order], jnp.zeros((CH,), jnp.int32)])
    h = x
    for l in range(NUM_LAYERS):
        parts, bases = _agg_call(h, ssrc, sdst)
        z = _mlp_call(h, parts, bases, W1s[l], b1s[l].reshape(1, D),
                      W2s[l], b2s[l].reshape(1, D))
        mean = jnp.mean(z, axis=0)
        var = jnp.var(z, axis=0)
        h = _bn_call(z, mean.reshape(1, D), var.reshape(1, D),
                     gammas[l].reshape(1, D), betas[l].reshape(1, D),
                     last=(l == NUM_LAYERS - 1))
    out = _head_call(h, batch.astype(jnp.int32).reshape(1, N),
                     pred_W, pred_b.reshape(1, NUM_TASKS))
    return out
